# baseline (speedup 1.0000x reference)
import functools

import jax
import jax.numpy as jnp
from jax.experimental import pallas as pl
from jax.experimental.pallas import tpu as pltpu

_dot = functools.partial(jnp.dot, preferred_element_type=jnp.float32)


def _nb_mean(x):
    return (jnp.roll(x, 1, 0) + jnp.roll(x, -1, 0)
            + jnp.roll(x, 1, 1) + jnp.roll(x, -1, 1)) * 0.25


def _sage(x, ws, wn, b, out_dtype=jnp.bfloat16):
    nx = x.shape[0]
    m = _nb_mean(x)
    y = _dot(x.reshape(nx * nx, -1), ws) + _dot(m.reshape(nx * nx, -1), wn) + b
    return jax.nn.relu(y).astype(out_dtype).reshape(nx, nx, -1)


def _pool(x):
    nx, ny, c = x.shape
    a = x.reshape(nx // 2, 2, ny, c)
    a = a[:, 0] + a[:, 1]
    b = a.reshape(nx // 2, ny // 2, 2, c)
    b = b[:, :, 0] + b[:, :, 1]
    return b * 0.25


def _upsample(x):
    nx, ny, c = x.shape
    u = jnp.broadcast_to(x[:, None], (nx, 2, ny, c)).reshape(2 * nx, ny, c)
    u = jnp.broadcast_to(u[:, :, None], (2 * nx, ny, 2, c)).reshape(2 * nx, 2 * ny, c)
    return u


def _unet_pair(x_ref,
               dc1ws, dc1wn, dc1b, dc2ws, dc2wn, dc2b,
               lc1ws, lc1wn, lc1b, lc2ws, lc2wn, lc2b,
               uc1ws_u, uc1ws_s, uc1wn_u, uc1wn_s, uc1b,
               uc2ws, uc2wn, uc2b,
               out_ref):
    h = out_ref.shape[-1]
    x = jnp.concatenate([x_ref[0], x_ref[1]], axis=-1)
    x = _sage(x, dc1ws[...], dc1wn[...], dc1b[...])
    x = _sage(x, dc2ws[...], dc2wn[...], dc2b[...])
    skip = x
    p = _pool(x)
    p = _sage(p, lc1ws[...], lc1wn[...], lc1b[...])
    p = _sage(p, lc2ws[...], lc2wn[...], lc2b[...])
    u = _upsample(p)
    nx = u.shape[0]
    uf = u.reshape(nx * nx, -1)
    sf = skip.reshape(nx * nx, -1)
    hs = _dot(uf, uc1ws_u[...]) + _dot(sf, uc1ws_s[...])
    hn = (_dot(uf, uc1wn_u[...]) + _dot(sf, uc1wn_s[...])).astype(jnp.bfloat16)
    y = jax.nn.relu(hs + _nb_mean(hn.reshape(nx, nx, -1)).astype(jnp.float32)
                    .reshape(nx * nx, -1) + uc1b[...])
    y = _sage(y.astype(jnp.bfloat16).reshape(nx, nx, -1),
              uc2ws[...], uc2wn[...], uc2b[...], out_dtype=jnp.float32)
    out_ref[0] = y[..., :h]
    out_ref[1] = y[..., h:]


def _diag2(w):
    ci, co = w.shape
    z = jnp.zeros_like(w)
    d = jnp.concatenate(
        [jnp.concatenate([w, z], axis=1), jnp.concatenate([z, w], axis=1)],
        axis=0)
    return d.astype(jnp.bfloat16)


def kernel(inputs,
           down_c1_ws, down_c1_wn, down_c1_b,
           down_c2_ws, down_c2_wn, down_c2_b,
           low_c1_ws, low_c1_wn, low_c1_b,
           low_c2_ws, low_c2_wn, low_c2_b,
           up_c1_ws, up_c1_wn, up_c1_b,
           up_c2_ws, up_c2_wn, up_c2_b):
    B, T, NX, NY, CIN = inputs.shape
    H = down_c1_ws.shape[1]
    S = B * T
    G = S // 2
    x = inputs.reshape(S, NX, NY, CIN).astype(jnp.bfloat16)

    def b2(b):
        return jnp.concatenate([b, b]).reshape(1, 2 * b.shape[0])

    weights = (
        _diag2(down_c1_ws), _diag2(down_c1_wn), b2(down_c1_b),
        _diag2(down_c2_ws), _diag2(down_c2_wn), b2(down_c2_b),
        _diag2(low_c1_ws), _diag2(low_c1_wn), b2(low_c1_b),
        _diag2(low_c2_ws), _diag2(low_c2_wn), b2(low_c2_b),
        _diag2(up_c1_ws[:H]), _diag2(up_c1_ws[H:]),
        _diag2(up_c1_wn[:H]), _diag2(up_c1_wn[H:]), b2(up_c1_b),
        _diag2(up_c2_ws), _diag2(up_c2_wn), b2(up_c2_b),
    )

    in_specs = [pl.BlockSpec((2, NX, NY, CIN), lambda i: (i, 0, 0, 0))]
    for w in weights:
        in_specs.append(pl.BlockSpec(w.shape, lambda i: (0,) * w.ndim))

    out = pl.pallas_call(
        _unet_pair,
        grid=(G,),
        in_specs=in_specs,
        out_specs=pl.BlockSpec((2, NX, NY, H), lambda i: (i, 0, 0, 0)),
        out_shape=jax.ShapeDtypeStruct((S, NX, NY, H), jnp.float32),
        compiler_params=pltpu.CompilerParams(
            vmem_limit_bytes=64 * 1024 * 1024),
    )(x, *weights)
    return out.reshape(B, T, NX, NY, H)


# speedup vs baseline: 1.2875x; 1.2875x over previous
import jax
import jax.numpy as jnp
from jax.experimental import pallas as pl
from jax.experimental.pallas import tpu as pltpu


def _nb_mean(x, jshift=1):
    return (jnp.roll(x, 1, 0) + jnp.roll(x, -1, 0)
            + jnp.roll(x, jshift, 1) + jnp.roll(x, -jshift, 1)) * 0.25


def _sage(x, ws, wn, b, jshift=1):
    n = x.shape[0] * x.shape[1]
    m = _nb_mean(x, jshift)
    y = x.reshape(n, -1) @ ws + m.reshape(n, -1) @ wn + b
    return jax.nn.relu(y).reshape(x.shape[0], x.shape[1], -1)


def _pool_rep(x):
    nx2, ny2, c = x.shape
    a = x.reshape(nx2 // 2, 2, ny2, c)
    a = a[:, 0] + a[:, 1]
    s = a + jnp.roll(a, -1, axis=1)
    evenj = (jax.lax.broadcasted_iota(jnp.int32, (1, ny2, 1), 1) % 2) == 0
    return 0.25 * jnp.where(evenj, s, jnp.roll(s, 1, axis=1))


def _upsample_rep(x):
    nx, ny, c = x.shape
    return jnp.broadcast_to(x[:, None], (nx, 2, ny, c)).reshape(2 * nx, ny, c)


def _unet_pair(x_ref,
               dc1ws, dc1wn, dc1b, dc2ws, dc2wn, dc2b,
               lc1ws, lc1wn, lc1b, lc2ws, lc2wn, lc2b,
               uc1ws_u, uc1ws_s, uc1wn_u, uc1wn_s, uc1b,
               uc2ws, uc2wn, uc2b,
               out_ref):
    h = out_ref.shape[-1]
    x = x_ref[0]
    x = _sage(x, dc1ws[...], dc1wn[...], dc1b[...])
    x = _sage(x, dc2ws[...], dc2wn[...], dc2b[...])
    skip = x
    p = _pool_rep(x)
    p = _sage(p, lc1ws[...], lc1wn[...], lc1b[...], jshift=2)
    p = _sage(p, lc2ws[...], lc2wn[...], lc2b[...], jshift=2)
    u = _upsample_rep(p)
    nx = u.shape[0]
    uf = u.reshape(nx * nx, -1)
    sf = skip.reshape(nx * nx, -1)
    hs = uf @ uc1ws_u[...] + sf @ uc1ws_s[...]
    hn = (uf @ uc1wn_u[...] + sf @ uc1wn_s[...]).reshape(nx, nx, -1)
    y = jax.nn.relu(hs.reshape(nx, nx, -1) + _nb_mean(hn) + uc1b[...])
    y = _sage(y, uc2ws[...], uc2wn[...], uc2b[...])
    out_ref[0] = y[..., :h]
    out_ref[1] = y[..., h:]


def _diag2(w):
    ci, co = w.shape
    z = jnp.zeros_like(w)
    return jnp.concatenate(
        [jnp.concatenate([w, z], axis=1), jnp.concatenate([z, w], axis=1)],
        axis=0)


def kernel(inputs,
           down_c1_ws, down_c1_wn, down_c1_b,
           down_c2_ws, down_c2_wn, down_c2_b,
           low_c1_ws, low_c1_wn, low_c1_b,
           low_c2_ws, low_c2_wn, low_c2_b,
           up_c1_ws, up_c1_wn, up_c1_b,
           up_c2_ws, up_c2_wn, up_c2_b):
    B, T, NX, NY, CIN = inputs.shape
    H = down_c1_ws.shape[1]
    S = B * T
    G = S // 2
    x = (inputs.reshape(G, 2, NX, NY, CIN)
         .transpose(0, 2, 3, 1, 4).reshape(G, NX, NY, 2 * CIN))

    def b2(b):
        return jnp.concatenate([b, b]).reshape(1, 2 * b.shape[0])

    weights = (
        _diag2(down_c1_ws), _diag2(down_c1_wn), b2(down_c1_b),
        _diag2(down_c2_ws), _diag2(down_c2_wn), b2(down_c2_b),
        _diag2(low_c1_ws), _diag2(low_c1_wn), b2(low_c1_b),
        _diag2(low_c2_ws), _diag2(low_c2_wn), b2(low_c2_b),
        _diag2(up_c1_ws[:H]), _diag2(up_c1_ws[H:]),
        _diag2(up_c1_wn[:H]), _diag2(up_c1_wn[H:]), b2(up_c1_b),
        _diag2(up_c2_ws), _diag2(up_c2_wn), b2(up_c2_b),
    )

    in_specs = [pl.BlockSpec((1, NX, NY, 2 * CIN), lambda i: (i, 0, 0, 0))]
    for w in weights:
        in_specs.append(pl.BlockSpec(w.shape, lambda i: (0,) * w.ndim))

    out = pl.pallas_call(
        _unet_pair,
        grid=(G,),
        in_specs=in_specs,
        out_specs=pl.BlockSpec((2, NX, NY, H), lambda i: (i, 0, 0, 0)),
        out_shape=jax.ShapeDtypeStruct((S, NX, NY, H), jnp.float32),
        compiler_params=pltpu.CompilerParams(
            vmem_limit_bytes=64 * 1024 * 1024),
    )(x, *weights)
    return out.reshape(B, T, NX, NY, H)
